# baseline (speedup 1.0000x reference)
import functools

import jax
import jax.numpy as jnp
from jax import lax
from jax.experimental import pallas as pl
from jax.experimental.pallas import tpu as pltpu
from jax.experimental.pallas import tpu_sc as plsc

VOCAB = 100000
EMBED = 128
HIDDEN = 1024
OUT = 256
BATCH = 16384
SEQ = 20

NC = 2
NS = 16
L = 16
NW = NC * NS

SEGB = 16
SEG = SEGB * SEQ
RING = 2
EV = EMBED // L

NCHUNKS = 1


def _sc_pool(x_flat, table, batch):
    bpw = batch // NW
    ipw = bpw * SEQ
    nseg = bpw // SEGB
    mesh = plsc.VectorSubcoreMesh(
        core_axis_name="c", subcore_axis_name="s", num_cores=NC,
        num_subcores=NS)

    @functools.partial(
        pl.kernel,
        out_type=jax.ShapeDtypeStruct((batch, EMBED), jnp.float32),
        mesh=mesh,
        scratch_types=[
            pltpu.VMEM((ipw,), jnp.int32),
            pltpu.VMEM((RING * SEG, EMBED), jnp.float32),
            pltpu.VMEM((RING * SEGB, EMBED), jnp.float32),
            pltpu.SemaphoreType.DMA,
            pltpu.SemaphoreType.DMA,
        ],
        compiler_params=pltpu.CompilerParams(needs_layout_passes=False),
    )
    def k(x_hbm, table_hbm, out_hbm, idx_v, rows_v, pooled_v, gsem, osem):
        wid = lax.axis_index("s") * NC + lax.axis_index("c")

        pltpu.sync_copy(
            x_hbm.at[pl.ds(pl.multiple_of(wid * ipw, 8), ipw)], idx_v)

        def fire_gather(s):
            start = pl.multiple_of(s * SEG, 8)
            slot = pl.multiple_of(lax.rem(s, RING) * SEG, 8)
            pltpu.async_copy(
                table_hbm.at[idx_v.at[pl.ds(start, SEG)]],
                rows_v.at[pl.ds(slot, SEG)],
                gsem,
            )

        for r in range(RING):
            fire_gather(r)

        def seg_body(s, carry):
            slot = lax.rem(s, RING)
            rbase = pl.multiple_of(slot * SEG, 8)
            pbase = pl.multiple_of(lax.rem(s, RING) * SEGB, 8)
            pltpu.make_async_copy(
                table_hbm.at[pl.ds(0, SEG)],
                rows_v.at[pl.ds(rbase, SEG)],
                gsem,
            ).wait()
            @pl.when(s >= RING)
            def _():
                pltpu.make_async_copy(
                    pooled_v.at[pl.ds(pbase, SEGB)],
                    out_hbm.at[pl.ds(0, SEGB)],
                    osem,
                ).wait()

            def pool_body(b, carry2):
                r0 = rbase + b * SEQ
                for d in range(EV):
                    col = pl.ds(d * L, L)
                    vals = [rows_v[r0 + j, col] for j in range(SEQ)]
                    while len(vals) > 1:
                        nxt = [vals[i] + vals[i + 1]
                               for i in range(0, len(vals) - 1, 2)]
                        if len(vals) % 2:
                            nxt.append(vals[-1])
                        vals = nxt
                    pooled_v[pbase + b, col] = vals[0]
                return carry2

            lax.fori_loop(0, SEGB, pool_body, 0)
            pltpu.async_copy(
                pooled_v.at[pl.ds(pbase, SEGB)],
                out_hbm.at[pl.ds(
                    pl.multiple_of(wid * bpw + s * SEGB, 8), SEGB)],
                osem,
            )

            @pl.when(s < nseg - RING)
            def _():
                fire_gather(s + RING)

            return carry

        lax.fori_loop(0, nseg, seg_body, 0)

        for r in range(RING):
            pltpu.make_async_copy(
                pooled_v.at[pl.ds(r * SEGB, SEGB)],
                out_hbm.at[pl.ds(0, SEGB)],
                osem,
            ).wait()

    return k(x_flat, table)


def _mlp(s, W1p, b1, W2, b2, batch):
    BB = 2048

    def body(s_ref, w1_ref, b1_ref, w2_ref, b2_ref, o_ref):
        sv = jax.nn.sigmoid(s_ref[...]).astype(jnp.bfloat16)
        h = jnp.dot(sv, w1_ref[...],
                    preferred_element_type=jnp.float32) + b1_ref[...]
        h = jax.nn.sigmoid(h).astype(jnp.bfloat16)
        o_ref[...] = jnp.dot(h, w2_ref[...],
                             preferred_element_type=jnp.float32) + b2_ref[...]

    return pl.pallas_call(
        body,
        grid=(batch // BB,),
        in_specs=[
            pl.BlockSpec((BB, EMBED), lambda i: (i, 0)),
            pl.BlockSpec((EMBED, HIDDEN), lambda i: (0, 0)),
            pl.BlockSpec((1, HIDDEN), lambda i: (0, 0)),
            pl.BlockSpec((HIDDEN, OUT), lambda i: (0, 0)),
            pl.BlockSpec((1, OUT), lambda i: (0, 0)),
        ],
        out_specs=pl.BlockSpec((BB, OUT), lambda i: (i, 0)),
        out_shape=jax.ShapeDtypeStruct((batch, OUT), jnp.float32),
    )(s, W1p, b1.reshape(1, HIDDEN), W2, b2.reshape(1, OUT))


def kernel(x, table, W1, b1, W2, b2):
    W1b = W1.astype(jnp.bfloat16)
    W2b = W2.astype(jnp.bfloat16)
    x_flat = x.reshape(BATCH * SEQ)
    cb = BATCH // NCHUNKS
    if NCHUNKS == 1:
        return _mlp(_sc_pool(x_flat, table, BATCH), W1b, b1, W2b, b2, BATCH)
    ss = [_sc_pool(x_flat[i * cb * SEQ:(i + 1) * cb * SEQ], table, cb)
          for i in range(NCHUNKS)]
    outs = [_mlp(s, W1b, b1, W2b, b2, cb) for s in ss]
    return jnp.concatenate(outs, axis=0)


# speedup vs baseline: 1.0307x; 1.0307x over previous
import functools

import jax
import jax.numpy as jnp
from jax import lax
from jax.experimental import pallas as pl
from jax.experimental.pallas import tpu as pltpu
from jax.experimental.pallas import tpu_sc as plsc

VOCAB = 100000
EMBED = 128
HIDDEN = 1024
OUT = 256
BATCH = 16384
SEQ = 20

NC = 2
NS = 16
L = 16
NW = NC * NS

SEGB = 8
SEG = SEGB * SEQ
RING = 4
EV = EMBED // L

NCHUNKS = 1


def _sc_pool(x_flat, table, batch):
    bpw = batch // NW
    ipw = bpw * SEQ
    nseg = bpw // SEGB
    mesh = plsc.VectorSubcoreMesh(
        core_axis_name="c", subcore_axis_name="s", num_cores=NC,
        num_subcores=NS)

    @functools.partial(
        pl.kernel,
        out_type=jax.ShapeDtypeStruct((batch, EMBED), jnp.float32),
        mesh=mesh,
        scratch_types=[
            pltpu.VMEM((ipw,), jnp.int32),
            pltpu.VMEM((RING * SEG, EMBED), jnp.float32),
            pltpu.VMEM((RING * SEGB, EMBED), jnp.float32),
            pltpu.SemaphoreType.DMA,
            pltpu.SemaphoreType.DMA,
        ],
        compiler_params=pltpu.CompilerParams(needs_layout_passes=False),
    )
    def k(x_hbm, table_hbm, out_hbm, idx_v, rows_v, pooled_v, gsem, osem):
        wid = lax.axis_index("s") * NC + lax.axis_index("c")

        pltpu.sync_copy(
            x_hbm.at[pl.ds(pl.multiple_of(wid * ipw, 8), ipw)], idx_v)

        def fire_gather(s):
            start = pl.multiple_of(s * SEG, 8)
            slot = pl.multiple_of(lax.rem(s, RING) * SEG, 8)
            pltpu.async_copy(
                table_hbm.at[idx_v.at[pl.ds(start, SEG)]],
                rows_v.at[pl.ds(slot, SEG)],
                gsem,
            )

        for r in range(RING):
            fire_gather(r)

        def seg_body(s, carry):
            slot = lax.rem(s, RING)
            rbase = pl.multiple_of(slot * SEG, 8)
            pbase = pl.multiple_of(lax.rem(s, RING) * SEGB, 8)
            pltpu.make_async_copy(
                table_hbm.at[pl.ds(0, SEG)],
                rows_v.at[pl.ds(rbase, SEG)],
                gsem,
            ).wait()
            @pl.when(s >= RING)
            def _():
                pltpu.make_async_copy(
                    pooled_v.at[pl.ds(pbase, SEGB)],
                    out_hbm.at[pl.ds(0, SEGB)],
                    osem,
                ).wait()

            def pool_body(b, carry2):
                r0 = rbase + b * SEQ
                for d in range(EV):
                    col = pl.ds(d * L, L)
                    vals = [rows_v[r0 + j, col] for j in range(SEQ)]
                    while len(vals) > 1:
                        nxt = [vals[i] + vals[i + 1]
                               for i in range(0, len(vals) - 1, 2)]
                        if len(vals) % 2:
                            nxt.append(vals[-1])
                        vals = nxt
                    pooled_v[pbase + b, col] = vals[0]
                return carry2

            lax.fori_loop(0, SEGB, pool_body, 0)
            pltpu.async_copy(
                pooled_v.at[pl.ds(pbase, SEGB)],
                out_hbm.at[pl.ds(
                    pl.multiple_of(wid * bpw + s * SEGB, 8), SEGB)],
                osem,
            )

            @pl.when(s < nseg - RING)
            def _():
                fire_gather(s + RING)

            return carry

        lax.fori_loop(0, nseg, seg_body, 0)

        for r in range(RING):
            pltpu.make_async_copy(
                pooled_v.at[pl.ds(r * SEGB, SEGB)],
                out_hbm.at[pl.ds(0, SEGB)],
                osem,
            ).wait()

    return k(x_flat, table)


def _mlp(s, W1p, b1, W2, b2, batch):
    BB = 4096

    def body(s_ref, w1_ref, b1_ref, w2_ref, b2_ref, o_ref):
        sv = jax.nn.sigmoid(s_ref[...]).astype(jnp.bfloat16)
        h = jnp.dot(sv, w1_ref[...],
                    preferred_element_type=jnp.float32) + b1_ref[...]
        h = jax.nn.sigmoid(h).astype(jnp.bfloat16)
        o_ref[...] = jnp.dot(h, w2_ref[...],
                             preferred_element_type=jnp.float32) + b2_ref[...]

    return pl.pallas_call(
        body,
        grid=(batch // BB,),
        in_specs=[
            pl.BlockSpec((BB, EMBED), lambda i: (i, 0)),
            pl.BlockSpec((EMBED, HIDDEN), lambda i: (0, 0)),
            pl.BlockSpec((1, HIDDEN), lambda i: (0, 0)),
            pl.BlockSpec((HIDDEN, OUT), lambda i: (0, 0)),
            pl.BlockSpec((1, OUT), lambda i: (0, 0)),
        ],
        out_specs=pl.BlockSpec((BB, OUT), lambda i: (i, 0)),
        out_shape=jax.ShapeDtypeStruct((batch, OUT), jnp.float32),
    )(s, W1p, b1.reshape(1, HIDDEN), W2, b2.reshape(1, OUT))


def kernel(x, table, W1, b1, W2, b2):
    W1b = W1.astype(jnp.bfloat16)
    W2b = W2.astype(jnp.bfloat16)
    x_flat = x.reshape(BATCH * SEQ)
    cb = BATCH // NCHUNKS
    if NCHUNKS == 1:
        return _mlp(_sc_pool(x_flat, table, BATCH), W1b, b1, W2b, b2, BATCH)
    ss = [_sc_pool(x_flat[i * cb * SEQ:(i + 1) * cb * SEQ], table, cb)
          for i in range(NCHUNKS)]
    outs = [_mlp(s, W1b, b1, W2b, b2, cb) for s in ss]
    return jnp.concatenate(outs, axis=0)


# speedup vs baseline: 1.0648x; 1.0330x over previous
import functools

import jax
import jax.numpy as jnp
from jax import lax
from jax.experimental import pallas as pl
from jax.experimental.pallas import tpu as pltpu
from jax.experimental.pallas import tpu_sc as plsc

VOCAB = 100000
EMBED = 128
HIDDEN = 1024
OUT = 256
BATCH = 16384
SEQ = 20

NC = 2
NS = 16
L = 16
NW = NC * NS

SEGB = 8
SEG = SEGB * SEQ
RING = 4
EV = EMBED // L

NCHUNKS = 1


def _sc_pool(x_flat, table, batch):
    bpw = batch // NW
    ipw = bpw * SEQ
    nseg = bpw // SEGB
    mesh = plsc.VectorSubcoreMesh(
        core_axis_name="c", subcore_axis_name="s", num_cores=NC,
        num_subcores=NS)

    @functools.partial(
        pl.kernel,
        out_type=jax.ShapeDtypeStruct((batch, EMBED), jnp.float32),
        mesh=mesh,
        scratch_types=[
            pltpu.VMEM((ipw,), jnp.int32),
            pltpu.VMEM((RING * SEG, EMBED), jnp.float32),
            pltpu.VMEM((RING * SEGB, EMBED), jnp.float32),
            pltpu.SemaphoreType.DMA,
            pltpu.SemaphoreType.DMA,
        ],
        compiler_params=pltpu.CompilerParams(needs_layout_passes=False),
    )
    def k(x_hbm, table_hbm, out_hbm, idx_v, rows_v, pooled_v, gsem, osem):
        wid = lax.axis_index("s") * NC + lax.axis_index("c")

        pltpu.sync_copy(
            x_hbm.at[pl.ds(pl.multiple_of(wid * ipw, 8), ipw)], idx_v)

        def fire_gather(s):
            start = pl.multiple_of(s * SEG, 8)
            slot = pl.multiple_of(lax.rem(s, RING) * SEG, 8)
            pltpu.async_copy(
                table_hbm.at[idx_v.at[pl.ds(start, SEG)]],
                rows_v.at[pl.ds(slot, SEG)],
                gsem,
            )

        for r in range(RING):
            fire_gather(r)

        def seg_body(s, carry):
            slot = lax.rem(s, RING)
            rbase = pl.multiple_of(slot * SEG, 8)
            pbase = pl.multiple_of(lax.rem(s, RING) * SEGB, 8)
            pltpu.make_async_copy(
                table_hbm.at[pl.ds(0, SEG)],
                rows_v.at[pl.ds(rbase, SEG)],
                gsem,
            ).wait()
            @pl.when(s >= RING)
            def _():
                pltpu.make_async_copy(
                    pooled_v.at[pl.ds(pbase, SEGB)],
                    out_hbm.at[pl.ds(0, SEGB)],
                    osem,
                ).wait()

            def pool_body(b, carry2):
                r0 = rbase + b * SEQ
                for d in range(EV):
                    col = pl.ds(d * L, L)
                    vals = [rows_v[r0 + j, col] for j in range(SEQ)]
                    while len(vals) > 1:
                        nxt = [vals[i] + vals[i + 1]
                               for i in range(0, len(vals) - 1, 2)]
                        if len(vals) % 2:
                            nxt.append(vals[-1])
                        vals = nxt
                    pooled_v[pbase + b, col] = vals[0]
                return carry2

            lax.fori_loop(0, SEGB, pool_body, 0)
            pltpu.async_copy(
                pooled_v.at[pl.ds(pbase, SEGB)],
                out_hbm.at[pl.ds(
                    pl.multiple_of(wid * bpw + s * SEGB, 8), SEGB)],
                osem,
            )

            @pl.when(s < nseg - RING)
            def _():
                fire_gather(s + RING)

            return carry

        lax.fori_loop(0, nseg, seg_body, 0)

        for r in range(RING):
            pltpu.make_async_copy(
                pooled_v.at[pl.ds(r * SEGB, SEGB)],
                out_hbm.at[pl.ds(0, SEGB)],
                osem,
            ).wait()

    return k(x_flat, table)


def _mlp(s, W1p, b1, W2, b2, batch):
    BB = 4096

    def body(s_ref, w1_ref, b1_ref, w2_ref, b2_ref, o_ref):
        sv = (0.5 + 0.5 * jnp.tanh(0.5 * s_ref[...])).astype(jnp.bfloat16)
        h = jnp.dot(sv, w1_ref[...],
                    preferred_element_type=jnp.float32) + b1_ref[...]
        h = (0.5 + 0.5 * jnp.tanh(0.5 * h)).astype(jnp.bfloat16)
        o_ref[...] = jnp.dot(h, w2_ref[...],
                             preferred_element_type=jnp.float32) + b2_ref[...]

    return pl.pallas_call(
        body,
        grid=(batch // BB,),
        in_specs=[
            pl.BlockSpec((BB, EMBED), lambda i: (i, 0)),
            pl.BlockSpec((EMBED, HIDDEN), lambda i: (0, 0)),
            pl.BlockSpec((1, HIDDEN), lambda i: (0, 0)),
            pl.BlockSpec((HIDDEN, OUT), lambda i: (0, 0)),
            pl.BlockSpec((1, OUT), lambda i: (0, 0)),
        ],
        out_specs=pl.BlockSpec((BB, OUT), lambda i: (i, 0)),
        out_shape=jax.ShapeDtypeStruct((batch, OUT), jnp.float32),
    )(s, W1p, b1.reshape(1, HIDDEN), W2, b2.reshape(1, OUT))


def kernel(x, table, W1, b1, W2, b2):
    W1b = W1.astype(jnp.bfloat16)
    W2b = W2.astype(jnp.bfloat16)
    x_flat = x.reshape(BATCH * SEQ)
    cb = BATCH // NCHUNKS
    if NCHUNKS == 1:
        return _mlp(_sc_pool(x_flat, table, BATCH), W1b, b1, W2b, b2, BATCH)
    ss = [_sc_pool(x_flat[i * cb * SEQ:(i + 1) * cb * SEQ], table, cb)
          for i in range(NCHUNKS)]
    outs = [_mlp(s, W1b, b1, W2b, b2, cb) for s in ss]
    return jnp.concatenate(outs, axis=0)
